# Initial kernel scaffold; baseline (speedup 1.0000x reference)
import jax
import jax.numpy as jnp
from jax.experimental import pallas as pl


def kernel(x, indices):
    raise NotImplementedError("write your pallas kernel here")


# speedup vs baseline: 9.7256x; 9.7256x over previous
import functools

import jax
import jax.numpy as jnp
from jax import lax
from jax.experimental import pallas as pl
from jax.experimental.pallas import tpu as pltpu
from jax.experimental.pallas import tpu_sc as plsc

_B, _N, _C = 4, 786432, 16
_NW = 32
_CH = 2048
_PER_W = _N // _NW
_NCHUNK = _PER_W // _CH
_GSUB = 128
_L = 16


def _order_map_kernel(x_hbm, idx_hbm, out_hbm, raw_v, cl0, cl1, cl2, cl3,
                      rows_v, sem):
    cls = (cl0, cl1, cl2, cl3)
    wid = lax.axis_index("s") * 2 + lax.axis_index("c")
    base_w = wid * _PER_W

    def chunk_body(ci, _):
        base = base_w + ci * _CH
        pltpu.sync_copy(idx_hbm.at[pl.ds(base, _CH)], raw_v)

        def xform(g, acc):
            iv = raw_v[pl.ds(g * _L, _L)]
            cl = jnp.minimum(iv, _N - 1)
            for b in range(_B):
                cls[b][pl.ds(g * _L, _L)] = cl + b * _N
            return acc | jnp.where(iv == _N, 1, 0).astype(jnp.int32)

        acc = lax.fori_loop(0, _CH // _L, xform,
                            jnp.zeros((_L,), jnp.int32))
        anybad = plsc.all_reduce_population_count(acc > 0)[0] > 0

        zeros = jnp.zeros((_L,), jnp.float32)
        iota = lax.iota(jnp.int32, _L)

        for b in range(_B):
            hs = [
                pltpu.async_copy(
                    x_hbm.at[cls[b].at[pl.ds(j * _GSUB, _GSUB)]],
                    rows_v.at[pl.ds(j * _GSUB, _GSUB)],
                    sem,
                )
                for j in range(_CH // _GSUB)
            ]
            for h in hs:
                h.wait()

            @pl.when(anybad)
            def _fixup():
                def fix(g, carry):
                    iv = raw_v[pl.ds(g * _L, _L)]
                    bad = iv == _N
                    rows_idx = g * _L + iota
                    for c in range(_C):
                        plsc.store_scatter(
                            rows_v,
                            [rows_idx, jnp.full((_L,), c, jnp.int32)],
                            zeros, mask=bad)
                    return carry

                lax.fori_loop(0, _CH // _L, fix, 0)

            pltpu.sync_copy(rows_v, out_hbm.at[pl.ds(b * _N + base, _CH)])
        return _

    lax.fori_loop(0, _NCHUNK, chunk_body, 0)


@jax.jit
def kernel(x, indices):
    mesh = plsc.VectorSubcoreMesh(core_axis_name="c", subcore_axis_name="s")
    run = functools.partial(
        pl.kernel,
        mesh=mesh,
        compiler_params=pltpu.CompilerParams(
            needs_layout_passes=False, use_tc_tiling_on_sc=False),
        out_type=jax.ShapeDtypeStruct((_B * _N, _C), jnp.float32),
        scratch_types=[
            pltpu.VMEM((_CH,), jnp.int32),
            pltpu.VMEM((_CH,), jnp.int32),
            pltpu.VMEM((_CH,), jnp.int32),
            pltpu.VMEM((_CH,), jnp.int32),
            pltpu.VMEM((_CH,), jnp.int32),
            pltpu.VMEM((_CH, _C), jnp.float32),
            pltpu.SemaphoreType.DMA,
        ],
    )(_order_map_kernel)
    out = run(x.reshape(_B * _N, _C), indices)
    return out.reshape(_B, _N, _C)


# speedup vs baseline: 9.7268x; 1.0001x over previous
import functools

import jax
import jax.numpy as jnp
from jax import lax
from jax.experimental import pallas as pl
from jax.experimental.pallas import tpu as pltpu
from jax.experimental.pallas import tpu_sc as plsc

_B, _N, _C = 4, 786432, 16
_NW = 32
_CH = 2048
_PER_W = _N // _NW
_NCHUNK = _PER_W // _CH
_GSUB = 128
_L = 16


def _order_map_kernel(x_hbm, idx_hbm, out_hbm, raw_v, cl_v, rows_v, sem):
    wid = lax.axis_index("s") * 2 + lax.axis_index("c")
    base_w = wid * _PER_W

    def chunk_body(ci, _):
        base = base_w + ci * _CH
        pltpu.sync_copy(idx_hbm.at[pl.ds(base, _CH)], raw_v)

        def xform(g, acc):
            iv = raw_v[pl.ds(g * _L, _L)]
            cl_v[pl.ds(g * _L, _L)] = jnp.minimum(iv, _N - 1)
            return acc | jnp.where(iv == _N, 1, 0).astype(jnp.int32)

        acc = lax.fori_loop(0, _CH // _L, xform,
                            jnp.zeros((_L,), jnp.int32))
        anybad = plsc.all_reduce_population_count(acc > 0)[0] > 0

        zeros = jnp.zeros((_L,), jnp.float32)
        iota = lax.iota(jnp.int32, _L)

        for b in range(_B):
            hs = [
                pltpu.async_copy(
                    x_hbm.at[b].at[cl_v.at[pl.ds(j * _GSUB, _GSUB)]],
                    rows_v.at[pl.ds(j * _GSUB, _GSUB)],
                    sem,
                )
                for j in range(_CH // _GSUB)
            ]
            for h in hs:
                h.wait()

            @pl.when(anybad)
            def _fixup():
                def fix(g, carry):
                    iv = raw_v[pl.ds(g * _L, _L)]
                    bad = iv == _N
                    rows_idx = g * _L + iota
                    for c in range(_C):
                        plsc.store_scatter(
                            rows_v,
                            [rows_idx, jnp.full((_L,), c, jnp.int32)],
                            zeros, mask=bad)
                    return carry

                lax.fori_loop(0, _CH // _L, fix, 0)

            pltpu.sync_copy(rows_v, out_hbm.at[b].at[pl.ds(base, _CH)])
        return _

    lax.fori_loop(0, _NCHUNK, chunk_body, 0)


@jax.jit
def kernel(x, indices):
    mesh = plsc.VectorSubcoreMesh(core_axis_name="c", subcore_axis_name="s")
    run = functools.partial(
        pl.kernel,
        mesh=mesh,
        compiler_params=pltpu.CompilerParams(
            needs_layout_passes=False, use_tc_tiling_on_sc=False),
        out_type=jax.ShapeDtypeStruct((_B, _N, _C), jnp.float32),
        scratch_types=[
            pltpu.VMEM((_CH,), jnp.int32),
            pltpu.VMEM((_CH,), jnp.int32),
            pltpu.VMEM((_CH, _C), jnp.float32),
            pltpu.SemaphoreType.DMA,
        ],
    )(_order_map_kernel)
    return run(x, indices)


# speedup vs baseline: 36.3248x; 3.7345x over previous
import functools

import jax
import jax.numpy as jnp
from jax import lax
from jax.experimental import pallas as pl
from jax.experimental.pallas import tpu as pltpu
from jax.experimental.pallas import tpu_sc as plsc

_B, _N, _C = 4, 786432, 16
_SUB = 16
_PPS = _N // _SUB
_CH = 24576
_GS = 128
_TAIL = 128
_NPLANES = _B * _C // 2


def _order_map_body(xt_hbm, idx_hbm, out_hbm, plane_sh, idx_v, outb_v,
                    zero_v, sem):
    cid = lax.axis_index("c")
    sid = lax.axis_index("s")

    pltpu.sync_copy(idx_hbm.at[pl.ds(sid * _PPS, _PPS)], idx_v)

    @pl.when(sid == 0)
    def _zero_tail():
        for k in range(_TAIL // 16):
            zero_v[pl.ds(k * 16, 16)] = jnp.zeros((16,), jnp.float32)
        pltpu.sync_copy(zero_v, plane_sh.at[pl.ds(_N, _TAIL)])

    def plane_body(p, _):
        g = cid * _NPLANES + p
        b = g // _C
        c = g % _C

        @pl.when(sid == 0)
        def _load():
            pltpu.sync_copy(xt_hbm.at[b, c], plane_sh.at[pl.ds(0, _N)])

        plsc.subcore_barrier()

        def chunk_body(ci, _):
            hs = [
                pltpu.async_copy(
                    plane_sh.at[idx_v.at[pl.ds(ci * _CH + j * _GS, _GS)]],
                    outb_v.at[pl.ds(j * _GS, _GS)],
                    sem,
                )
                for j in range(_CH // _GS)
            ]
            for h in hs:
                h.wait()
            pltpu.sync_copy(
                outb_v,
                out_hbm.at[b, c, pl.ds(sid * _PPS + ci * _CH, _CH)])
            return _

        lax.fori_loop(0, _PPS // _CH, chunk_body, 0)
        plsc.subcore_barrier()
        return _

    lax.fori_loop(0, _NPLANES, plane_body, 0)


@jax.jit
def kernel(x, indices):
    xt = jnp.transpose(x, (0, 2, 1))
    mesh = plsc.VectorSubcoreMesh(core_axis_name="c", subcore_axis_name="s")
    run = functools.partial(
        pl.kernel,
        mesh=mesh,
        compiler_params=pltpu.CompilerParams(
            needs_layout_passes=False, use_tc_tiling_on_sc=True),
        out_type=jax.ShapeDtypeStruct((_B, _C, _N), jnp.float32),
        scratch_types=[
            pltpu.VMEM_SHARED((_N + _TAIL,), jnp.float32),
            pltpu.VMEM((_PPS,), jnp.int32),
            pltpu.VMEM((_CH,), jnp.float32),
            pltpu.VMEM((_TAIL,), jnp.float32),
            pltpu.SemaphoreType.DMA,
        ],
    )(_order_map_body)
    out_t = run(xt, indices)
    return jnp.transpose(out_t, (0, 2, 1))
